# baseline (speedup 1.0000x reference)
import functools

import jax
import jax.numpy as jnp
import jax.scipy.linalg
from jax import lax
from jax.experimental import pallas as pl
from jax.experimental.pallas import tpu as pltpu
from jax.experimental.pallas import tpu_sc as plsc

_NUM_SC_CORES = 2
_NUM_SUBCORES = 16
_LANES = 16
_NW = _NUM_SC_CORES * _NUM_SUBCORES
_WIDE = 128


def _edge_proj8(ea8, wcat, bcat, widths, block_rows, interpret=False):
    E8, DE8 = ea8.shape
    offs = [0]
    for c in widths:
        offs.append(offs[-1] + 8 * c)

    def body(attr_ref, w_ref, b_ref, *o_refs):
        z = jnp.dot(attr_ref[...], w_ref[...],
                    preferred_element_type=jnp.float32) + b_ref[...]
        for i, o in enumerate(o_refs):
            o[...] = z[:, offs[i]:offs[i + 1]]

    in_specs = [
        pl.BlockSpec((block_rows, DE8), lambda i: (i, 0)),
        pl.BlockSpec(wcat.shape, lambda i: (0, 0)),
        pl.BlockSpec(bcat.shape, lambda i: (0, 0)),
    ]
    out_specs = [
        pl.BlockSpec((block_rows, 8 * c), lambda i: (i, 0)) for c in widths
    ]
    outs = [jax.ShapeDtypeStruct((E8, 8 * c), jnp.float32) for c in widths]
    return pl.pallas_call(
        body, grid=(E8 // block_rows,), in_specs=in_specs,
        out_specs=out_specs, out_shape=outs, interpret=interpret,
    )(ea8, wcat, bcat)


def _sc_edge_aggregate(h_pad, src, dst, e2, c_real, interpret=False):
    N = h_pad.shape[0]
    E = src.shape[0]
    EPW = E // _NW
    K = 16 if e2.shape[1] > 512 else 40
    assert EPW % K == 0
    NCH = EPW // K
    RE = K // 8
    REB = (RE + 15) // 8 * 8
    WE = e2.shape[1]
    assert WE == 8 * c_real and K % 8 == 0
    assert e2.shape[0] >= E // 8 + REB - RE
    SR = (N // (8 * _NUM_SUBCORES)) * 8
    TAIL = N - SR * _NUM_SUBCORES
    zeros = jnp.zeros((max(SR, TAIL), _WIDE), jnp.float32)
    mesh = plsc.VectorSubcoreMesh(
        core_axis_name="c", subcore_axis_name="s",
        num_cores=_NUM_SC_CORES, num_subcores=_NUM_SUBCORES)

    NB = 4
    assert NCH % NB >= 1 and NCH >= 3 * NB

    @functools.partial(
        pl.kernel,
        out_type=jax.ShapeDtypeStruct((_NUM_SC_CORES * N, _WIDE), jnp.float32),
        mesh=mesh,
        scratch_types=(
            [pltpu.VMEM((K,), jnp.int32) for _ in range(2 * NB)]
            + [pltpu.VMEM((K, _WIDE), jnp.float32) for _ in range(NB)]
            + [pltpu.VMEM((REB, WE), jnp.float32) for _ in range(2)]
            + [pltpu.VMEM_SHARED((N, _WIDE), jnp.float32)]
            + [pltpu.SemaphoreType.DMA for _ in range(3 * NB + 2)]
        ),
        interpret=interpret,
    )
    def body(h_hbm, src_hbm, dst_hbm, e_hbm, z_hbm, out_hbm, *scratch):
        srcs = scratch[0:NB]
        dsts = scratch[NB:2 * NB]
        rowss = scratch[2 * NB:3 * NB]
        evs = scratch[3 * NB:3 * NB + 2]
        acc_sh = scratch[3 * NB + 2]
        lsem = scratch[3 * NB + 3:4 * NB + 3]
        gsem = scratch[4 * NB + 3:5 * NB + 3]
        ssem = scratch[5 * NB + 3:6 * NB + 3]
        esem = scratch[6 * NB + 3:6 * NB + 5]
        cid = lax.axis_index("c")
        sid = lax.axis_index("s")
        srow = pl.multiple_of(sid * SR, 8)
        orow = pl.multiple_of(cid * N + sid * SR, 8)
        pltpu.sync_copy(z_hbm.at[pl.ds(0, SR), :],
                        acc_sh.at[pl.ds(srow, SR), :])
        if TAIL:
            @pl.when(sid == 0)
            def _():
                pltpu.sync_copy(z_hbm.at[pl.ds(0, TAIL), :],
                                acc_sh.at[pl.ds(N - TAIL, TAIL), :])
        plsc.subcore_barrier()
        base = cid * (E // _NUM_SC_CORES) + sid * EPW

        def load_descs(t, j):
            eb = pl.multiple_of(base + t * K, 8)
            return (
                pltpu.make_async_copy(src_hbm.at[pl.ds(eb, K)], srcs[j],
                                      lsem[j]),
                pltpu.make_async_copy(dst_hbm.at[pl.ds(eb, K)], dsts[j],
                                      lsem[j]),
            )

        def loads_start(t, j):
            for d in load_descs(t, j):
                d.start()

        def loads_wait(t, j):
            for d in load_descs(t, j):
                d.wait()

        def e_desc(t, je):
            er = (base + t * K) // 8
            er_al = pl.multiple_of((er // 8) * 8, 8)
            return pltpu.make_async_copy(e_hbm.at[pl.ds(er_al, REB), :],
                                         evs[je], esem[je])

        def gather_desc(j):
            return pltpu.make_async_copy(h_hbm.at[srcs[j]], rowss[j], gsem[j])

        def scatter_desc(j):
            return pltpu.make_async_copy(rowss[j], acc_sh.at[dsts[j]],
                                         ssem[j])

        def compute(j, je, eoff):
            rows_v, e_v = rowss[j], evs[je]

            @plsc.parallel_loop(0, RE, 1)
            def _(i):
                for p in range(8):
                    for q in range(c_real // _LANES):
                        s = pl.ds(q * _LANES, _LANES)
                        se = pl.ds(p * c_real + q * _LANES, _LANES)
                        rows_v[i * 8 + p, s] = jnp.maximum(
                            rows_v[i * 8 + p, s] + e_v[eoff + i, se], 0.0)

        def step(c, k, steady):
            j1 = (k + 1) % NB
            if steady or c + 1 <= NCH - 1:
                loads_wait(c + 1, j1)
                gather_desc(j1).start()
            gather_desc(k).wait()
            je = k % 2
            e_desc(c, je).wait()
            er = (base + c * K) // 8
            compute(k, je, er - (er // 8) * 8)
            if steady or c + 2 <= NCH - 1:
                e_desc(c + 2, je).start()
            scatter_desc(k).start(add=True)
            j3 = (k + NB - 1) % NB
            if steady or c + NB - 1 <= NCH - 1:
                if steady or c >= 1:
                    scatter_desc(j3).wait()
                loads_start(c + NB - 1, j3)

        for t in range(NB - 1):
            loads_start(t, t)
        e_desc(0, 0).start()
        e_desc(1, 1).start()
        loads_wait(0, 0)
        gather_desc(0).start()
        for c in range(NB):
            step(c, c % NB, False)

        GF = (NCH - 3) // NB

        def group(g, carry):
            for k in range(NB):
                step(NB * g + k, k, True)
            return carry

        lax.fori_loop(1, GF, group, 0)
        for c in range(NB * GF, NCH):
            step(c, c % NB, False)
        for m in range(NCH - NB, NCH):
            scatter_desc(m % NB).wait()
        plsc.subcore_barrier()
        pltpu.sync_copy(acc_sh.at[pl.ds(srow, SR), :],
                        out_hbm.at[pl.ds(orow, SR), :])
        if TAIL:
            @pl.when(sid == 0)
            def _():
                pltpu.sync_copy(
                    acc_sh.at[pl.ds(N - TAIL, TAIL), :],
                    out_hbm.at[pl.ds(pl.multiple_of(cid * N + N - TAIL, 8),
                                     TAIL), :])

    return body(h_pad, src, dst, e2, zeros)


def _node_mlp(h, agg, eps, w1t, b1, g1, be1, w2t, b2, bng, bnb, cin,
              trailing, out_pad, interpret=False):
    N = h.shape[0]
    DCo = w1t.shape[1]

    def body(h_ref, a_ref, eps_ref, w1_ref, b1_ref, g1_ref, be1_ref,
             w2_ref, b2_ref, bg_ref, bb_ref, o_ref):
        hv = h_ref[...][:, :cin]
        a = a_ref[0][:, :cin] + a_ref[1][:, :cin]
        z = (1.0 + eps_ref[0, 0]) * hv + a
        z = jnp.dot(z, w1_ref[...], preferred_element_type=jnp.float32) + b1_ref[...]
        m = jnp.mean(z, axis=0, keepdims=True)
        v = jnp.mean((z - m) ** 2, axis=0, keepdims=True)
        z = (z - m) / jnp.sqrt(v + 1e-5) * g1_ref[...] + be1_ref[...]
        z = jnp.where(z >= 0, z, 0.01 * z)
        z = jnp.dot(z, w2_ref[...], preferred_element_type=jnp.float32) + b2_ref[...]
        if trailing:
            m2 = jnp.mean(z, axis=0, keepdims=True)
            v2 = jnp.mean((z - m2) ** 2, axis=0, keepdims=True)
            z = (z - m2) / jnp.sqrt(v2 + 1e-5) * bg_ref[...] + bb_ref[...]
            z = jnp.where(z >= 0, z, 0.01 * z)
        if out_pad:
            o_ref[:, :DCo] = z
            o_ref[:, DCo:] = jnp.zeros((N, _WIDE - DCo), jnp.float32)
        else:
            o_ref[...] = z

    out_w = _WIDE if out_pad else DCo
    return pl.pallas_call(
        body, out_shape=jax.ShapeDtypeStruct((N, out_w), jnp.float32),
        interpret=interpret,
    )(h, agg, eps, w1t, b1, g1, be1, w2t, b2, bng, bnb)


def _final_mlp(h, mol_x, mw1t, mb1, mg, mbe, mw2t, mb2,
               fw1t, fb1, fg, fbe, fw2t, fb2, ch, interpret=False):
    N = h.shape[0]

    def body(h_ref, mx_ref, mw1_ref, mb1_ref, mg_ref, mbe_ref, mw2_ref,
             mb2_ref, fw1_ref, fb1_ref, fg_ref, fbe_ref, fw2_ref, fb2_ref,
             o_ref):
        xm = jnp.dot(mx_ref[...], mw1_ref[...],
                     preferred_element_type=jnp.float32) + mb1_ref[...]
        m = jnp.mean(xm, axis=0, keepdims=True)
        v = jnp.mean((xm - m) ** 2, axis=0, keepdims=True)
        xm = (xm - m) / jnp.sqrt(v + 1e-5) * mg_ref[...] + mbe_ref[...]
        xm = jnp.where(xm >= 0, xm, 0.01 * xm)
        xm = jnp.dot(xm, mw2_ref[...],
                     preferred_element_type=jnp.float32) + mb2_ref[...]
        fw1 = fw1_ref[...]
        y = (jnp.dot(h_ref[...][:, :ch], fw1[:ch],
                     preferred_element_type=jnp.float32)
             + jnp.dot(xm, fw1[ch:], preferred_element_type=jnp.float32)
             + fb1_ref[...])
        m = jnp.mean(y, axis=0, keepdims=True)
        v = jnp.mean((y - m) ** 2, axis=0, keepdims=True)
        y = (y - m) / jnp.sqrt(v + 1e-5) * fg_ref[...] + fbe_ref[...]
        y = jnp.where(y >= 0, y, 0.01 * y)
        y = jnp.dot(y, fw2_ref[...], preferred_element_type=jnp.float32) + fb2_ref[...]
        o_ref[...] = y

    return pl.pallas_call(
        body, out_shape=jax.ShapeDtypeStruct((N, 1), jnp.float32),
        interpret=interpret,
    )(h, mol_x, mw1t, mb1, mg, mbe, mw2t, mb2, fw1t, fb1, fg, fbe, fw2t, fb2)


def kernel(x, edge_index, edge_attr, mol_x, params):
    src = edge_index[0]
    dst = edge_index[1]
    N = x.shape[0]
    convs = params['convs']
    E = edge_attr.shape[0]
    DE = edge_attr.shape[1]
    widths = [L['lin_edge_W'].shape[0] for L in convs]

    E8 = E // 8
    BR8 = 2560
    E8P = ((E8 + 8 + BR8 - 1) // BR8) * BR8
    ea8 = jnp.pad(edge_attr.reshape(E8, 8 * DE), ((0, E8P - E8), (0, 0)))

    def blkdiag8(wt):
        return jax.scipy.linalg.block_diag(*([wt] * 8))

    e0 = _edge_proj8(ea8, blkdiag8(convs[0]['lin_edge_W'].T),
                     jnp.tile(convs[0]['lin_edge_b'], 8).reshape(1, -1),
                     [widths[0]], BR8)[0]
    wcat = jnp.concatenate(
        [blkdiag8(L['lin_edge_W'].T) for L in convs[1:]], axis=1)
    bcat = jnp.concatenate(
        [jnp.tile(L['lin_edge_b'], 8) for L in convs[1:]]).reshape(1, -1)
    es = [e0] + list(_edge_proj8(ea8, wcat, bcat, widths[1:], BR8))
    h = x
    for i, L in enumerate(convs):
        cin = widths[i]
        agg = _sc_edge_aggregate(h, src, dst, es[i], cin).reshape(2, N, _WIDE)
        h = _node_mlp(
            h, agg, L['eps'].reshape(1, 1),
            L['W1'].T, L['b1'].reshape(1, -1),
            L['g1'].reshape(1, -1), L['be1'].reshape(1, -1),
            L['W2'].T, L['b2'].reshape(1, -1),
            L['bn_g'].reshape(1, -1), L['bn_b'].reshape(1, -1),
            cin=cin, trailing=(i != len(convs) - 1),
            out_pad=(i != len(convs) - 1),
        )
    M, F = params['mol'], params['final']
    y = _final_mlp(
        h, mol_x,
        M['W1'].T, M['b1'].reshape(1, -1), M['g'].reshape(1, -1),
        M['be'].reshape(1, -1), M['W2'].T, M['b2'].reshape(1, -1),
        F['W1'].T, F['b1'].reshape(1, -1), F['g'].reshape(1, -1),
        F['be'].reshape(1, -1), F['W2'].T, F['b2'].reshape(1, -1),
        ch=convs[-1]['W2'].shape[0],
    )
    return y.reshape(-1)


# speedup vs baseline: 1.5488x; 1.5488x over previous
import functools

import jax
import jax.numpy as jnp
from jax import lax
from jax.experimental import pallas as pl
from jax.experimental.pallas import tpu as pltpu
from jax.experimental.pallas import tpu_sc as plsc

_NUM_SC_CORES = 2
_NUM_SUBCORES = 16
_LANES = 16
_NW = _NUM_SC_CORES * _NUM_SUBCORES
_WIDE = 128


def _edge_proj_l0(edge_attr, wt, b, block_rows, interpret=False):
    E, DE = edge_attr.shape
    C = wt.shape[1]

    def body(attr_ref, w_ref, b_ref, o_ref):
        o_ref[...] = jnp.dot(attr_ref[...], w_ref[...],
                             preferred_element_type=jnp.float32) + b_ref[...]

    return pl.pallas_call(
        body, grid=(E // block_rows,),
        in_specs=[
            pl.BlockSpec((block_rows, DE), lambda i: (i, 0)),
            pl.BlockSpec(wt.shape, lambda i: (0, 0)),
            pl.BlockSpec(b.shape, lambda i: (0, 0)),
        ],
        out_specs=pl.BlockSpec((block_rows, C), lambda i: (i, 0)),
        out_shape=jax.ShapeDtypeStruct((E, C), jnp.float32),
        interpret=interpret,
    )(edge_attr, wt, b)


def _edge_proj(ea2, wcat, bcat, widths, block_rows, interpret=False):
    E2, DE2 = ea2.shape
    offs = [0]
    for c in widths:
        offs.append(offs[-1] + 2 * c)

    def body(attr_ref, w_ref, b_ref, *o_refs):
        z = jnp.dot(attr_ref[...], w_ref[...],
                    preferred_element_type=jnp.float32) + b_ref[...]
        for i, o in enumerate(o_refs):
            o[...] = z[:, offs[i]:offs[i + 1]]

    in_specs = [
        pl.BlockSpec((block_rows, DE2), lambda i: (i, 0)),
        pl.BlockSpec(wcat.shape, lambda i: (0, 0)),
        pl.BlockSpec(bcat.shape, lambda i: (0, 0)),
    ]
    out_specs = [
        pl.BlockSpec((block_rows, 2 * c), lambda i: (i, 0)) for c in widths
    ]
    outs = [jax.ShapeDtypeStruct((E2, 2 * c), jnp.float32) for c in widths]
    return pl.pallas_call(
        body, grid=(E2 // block_rows,), in_specs=in_specs,
        out_specs=out_specs, out_shape=outs, interpret=interpret,
    )(ea2, wcat, bcat)


def _sc_edge_aggregate(h_pad, src, dst, e2, c_real, interpret=False):
    N = h_pad.shape[0]
    E = src.shape[0]
    EPW = E // _NW
    K = 40 if EPW % 40 == 0 else 8
    NCH = EPW // K
    packed = c_real < _WIDE
    KE = K // 2 if packed else K
    KEB = KE + 4 if packed else KE
    WE = e2.shape[1]
    assert WE == _WIDE
    SR = (N // (8 * _NUM_SUBCORES)) * 8
    TAIL = N - SR * _NUM_SUBCORES
    zeros = jnp.zeros((max(SR, TAIL), _WIDE), jnp.float32)
    mesh = plsc.VectorSubcoreMesh(
        core_axis_name="c", subcore_axis_name="s",
        num_cores=_NUM_SC_CORES, num_subcores=_NUM_SUBCORES)

    NB = 4
    assert NCH % NB >= 1 and NCH >= 3 * NB

    @functools.partial(
        pl.kernel,
        out_type=jax.ShapeDtypeStruct((_NUM_SC_CORES * N, _WIDE), jnp.float32),
        mesh=mesh,
        scratch_types=(
            [pltpu.VMEM((K,), jnp.int32) for _ in range(2 * NB)]
            + [pltpu.VMEM((K, _WIDE), jnp.float32) for _ in range(NB)]
            + [pltpu.VMEM((KEB, WE), jnp.float32) for _ in range(NB)]
            + [pltpu.VMEM_SHARED((N, _WIDE), jnp.float32)]
            + [pltpu.SemaphoreType.DMA for _ in range(3 * NB)]
        ),
        interpret=interpret,
    )
    def body(h_hbm, src_hbm, dst_hbm, e_hbm, z_hbm, out_hbm, *scratch):
        srcs = scratch[0:NB]
        dsts = scratch[NB:2 * NB]
        rowss = scratch[2 * NB:3 * NB]
        evs = scratch[3 * NB:4 * NB]
        acc_sh = scratch[4 * NB]
        lsem = scratch[4 * NB + 1:5 * NB + 1]
        gsem = scratch[5 * NB + 1:6 * NB + 1]
        ssem = scratch[6 * NB + 1:7 * NB + 1]
        cid = lax.axis_index("c")
        sid = lax.axis_index("s")
        srow = pl.multiple_of(sid * SR, 8)
        orow = pl.multiple_of(cid * N + sid * SR, 8)
        pltpu.sync_copy(z_hbm.at[pl.ds(0, SR), :],
                        acc_sh.at[pl.ds(srow, SR), :])
        if TAIL:
            @pl.when(sid == 0)
            def _():
                pltpu.sync_copy(z_hbm.at[pl.ds(0, TAIL), :],
                                acc_sh.at[pl.ds(N - TAIL, TAIL), :])
        plsc.subcore_barrier()
        base = cid * (E // _NUM_SC_CORES) + sid * EPW

        def load_descs(t, j):
            eb = pl.multiple_of(base + t * K, 8)
            if packed:
                er_al = pl.multiple_of(((eb // 2) // 8) * 8, 8)
            else:
                er_al = eb
            return (
                pltpu.make_async_copy(src_hbm.at[pl.ds(eb, K)], srcs[j],
                                      lsem[j]),
                pltpu.make_async_copy(dst_hbm.at[pl.ds(eb, K)], dsts[j],
                                      lsem[j]),
                pltpu.make_async_copy(e_hbm.at[pl.ds(er_al, KEB), :],
                                      evs[j], lsem[j]),
            )

        def loads_start(t, j):
            for d in load_descs(t, j):
                d.start()

        def loads_wait(t, j):
            for d in load_descs(t, j):
                d.wait()

        def gather_desc(j):
            return pltpu.make_async_copy(h_hbm.at[srcs[j]], rowss[j], gsem[j])

        def scatter_desc(j):
            return pltpu.make_async_copy(rowss[j], acc_sh.at[dsts[j]],
                                         ssem[j])

        def compute(j, eoff):
            rows_v, e_v = rowss[j], evs[j]

            if packed:
                @plsc.parallel_loop(0, KE, 1, unroll=4)
                def _(i):
                    for p in range(2):
                        for q in range(c_real // _LANES):
                            s = pl.ds(q * _LANES, _LANES)
                            se = pl.ds(p * c_real + q * _LANES, _LANES)
                            rows_v[i * 2 + p, s] = jnp.maximum(
                                rows_v[i * 2 + p, s] + e_v[eoff + i, se], 0.0)
            else:
                @plsc.parallel_loop(0, KE, 1, unroll=4)
                def _(i):
                    for q in range(c_real // _LANES):
                        s = pl.ds(q * _LANES, _LANES)
                        rows_v[i, s] = jnp.maximum(
                            rows_v[i, s] + e_v[i, s], 0.0)

        def step(c, k, steady):
            j1 = (k + 1) % NB
            if steady or c + 1 <= NCH - 1:
                loads_wait(c + 1, j1)
                gather_desc(j1).start()
            gather_desc(k).wait()
            if packed:
                er = (base + c * K) // 2
                compute(k, er - (er // 8) * 8)
            else:
                compute(k, 0)
            scatter_desc(k).start(add=True)
            j3 = (k + 3) % NB
            if steady or c + 3 <= NCH - 1:
                if steady or c >= 1:
                    scatter_desc(j3).wait()
                loads_start(c + 3, j3)

        for t in range(NB - 1):
            loads_start(t, t)
        loads_wait(0, 0)
        gather_desc(0).start()
        for c in range(NB):
            step(c, c % NB, False)

        GF = (NCH - 3) // NB

        def group(g, carry):
            for k in range(NB):
                step(NB * g + k, k, True)
            return carry

        lax.fori_loop(1, GF, group, 0)
        for c in range(NB * GF, NCH):
            step(c, c % NB, False)
        for m in range(NCH - 4, NCH):
            scatter_desc(m % NB).wait()
        plsc.subcore_barrier()
        pltpu.sync_copy(acc_sh.at[pl.ds(srow, SR), :],
                        out_hbm.at[pl.ds(orow, SR), :])
        if TAIL:
            @pl.when(sid == 0)
            def _():
                pltpu.sync_copy(
                    acc_sh.at[pl.ds(N - TAIL, TAIL), :],
                    out_hbm.at[pl.ds(pl.multiple_of(cid * N + N - TAIL, 8),
                                     TAIL), :])

    return body(h_pad, src, dst, e2, zeros)


def _node_mlp(h, agg, eps, w1t, b1, g1, be1, w2t, b2, bng, bnb, cin,
              trailing, out_pad, interpret=False):
    N = h.shape[0]
    DCo = w1t.shape[1]

    def body(h_ref, a_ref, eps_ref, w1_ref, b1_ref, g1_ref, be1_ref,
             w2_ref, b2_ref, bg_ref, bb_ref, o_ref):
        hv = h_ref[...][:, :cin]
        a = a_ref[0][:, :cin] + a_ref[1][:, :cin]
        z = (1.0 + eps_ref[0, 0]) * hv + a
        z = jnp.dot(z, w1_ref[...], preferred_element_type=jnp.float32) + b1_ref[...]
        m = jnp.mean(z, axis=0, keepdims=True)
        v = jnp.mean((z - m) ** 2, axis=0, keepdims=True)
        z = (z - m) / jnp.sqrt(v + 1e-5) * g1_ref[...] + be1_ref[...]
        z = jnp.where(z >= 0, z, 0.01 * z)
        z = jnp.dot(z, w2_ref[...], preferred_element_type=jnp.float32) + b2_ref[...]
        if trailing:
            m2 = jnp.mean(z, axis=0, keepdims=True)
            v2 = jnp.mean((z - m2) ** 2, axis=0, keepdims=True)
            z = (z - m2) / jnp.sqrt(v2 + 1e-5) * bg_ref[...] + bb_ref[...]
            z = jnp.where(z >= 0, z, 0.01 * z)
        if out_pad:
            o_ref[:, :DCo] = z
            o_ref[:, DCo:] = jnp.zeros((N, _WIDE - DCo), jnp.float32)
        else:
            o_ref[...] = z

    out_w = _WIDE if out_pad else DCo
    return pl.pallas_call(
        body, out_shape=jax.ShapeDtypeStruct((N, out_w), jnp.float32),
        interpret=interpret,
    )(h, agg, eps, w1t, b1, g1, be1, w2t, b2, bng, bnb)


def _final_mlp(h, mol_x, mw1t, mb1, mg, mbe, mw2t, mb2,
               fw1t, fb1, fg, fbe, fw2t, fb2, ch, interpret=False):
    N = h.shape[0]

    def body(h_ref, mx_ref, mw1_ref, mb1_ref, mg_ref, mbe_ref, mw2_ref,
             mb2_ref, fw1_ref, fb1_ref, fg_ref, fbe_ref, fw2_ref, fb2_ref,
             o_ref):
        xm = jnp.dot(mx_ref[...], mw1_ref[...],
                     preferred_element_type=jnp.float32) + mb1_ref[...]
        m = jnp.mean(xm, axis=0, keepdims=True)
        v = jnp.mean((xm - m) ** 2, axis=0, keepdims=True)
        xm = (xm - m) / jnp.sqrt(v + 1e-5) * mg_ref[...] + mbe_ref[...]
        xm = jnp.where(xm >= 0, xm, 0.01 * xm)
        xm = jnp.dot(xm, mw2_ref[...],
                     preferred_element_type=jnp.float32) + mb2_ref[...]
        fw1 = fw1_ref[...]
        y = (jnp.dot(h_ref[...][:, :ch], fw1[:ch],
                     preferred_element_type=jnp.float32)
             + jnp.dot(xm, fw1[ch:], preferred_element_type=jnp.float32)
             + fb1_ref[...])
        m = jnp.mean(y, axis=0, keepdims=True)
        v = jnp.mean((y - m) ** 2, axis=0, keepdims=True)
        y = (y - m) / jnp.sqrt(v + 1e-5) * fg_ref[...] + fbe_ref[...]
        y = jnp.where(y >= 0, y, 0.01 * y)
        y = jnp.dot(y, fw2_ref[...], preferred_element_type=jnp.float32) + fb2_ref[...]
        o_ref[...] = y

    return pl.pallas_call(
        body, out_shape=jax.ShapeDtypeStruct((N, 1), jnp.float32),
        interpret=interpret,
    )(h, mol_x, mw1t, mb1, mg, mbe, mw2t, mb2, fw1t, fb1, fg, fbe, fw2t, fb2)


def kernel(x, edge_index, edge_attr, mol_x, params):
    src = edge_index[0]
    dst = edge_index[1]
    N = x.shape[0]
    convs = params['convs']
    E = edge_attr.shape[0]
    DE = edge_attr.shape[1]
    ea2 = edge_attr.reshape(-1, 2 * DE)
    widths = [L['lin_edge_W'].shape[0] for L in convs]

    def blkdiag(wt):
        c = wt.shape[1]
        z = jnp.zeros((DE, c), jnp.float32)
        return jnp.concatenate(
            [jnp.concatenate([wt, z], 1), jnp.concatenate([z, wt], 1)], 0)

    be0 = 4000 if E % 4000 == 0 else E
    e0 = _edge_proj_l0(edge_attr, convs[0]['lin_edge_W'].T,
                       convs[0]['lin_edge_b'].reshape(1, -1), be0)
    wcat = jnp.concatenate(
        [blkdiag(L['lin_edge_W'].T) for L in convs[1:]], axis=1)
    bcat = jnp.concatenate(
        [jnp.concatenate([L['lin_edge_b']] * 2)
         for L in convs[1:]]).reshape(1, -1)
    E2 = E // 2
    block_rows = 1000 if E2 % 1000 == 0 else E2
    es = [e0] + list(_edge_proj(ea2, wcat, bcat, widths[1:], block_rows))
    h = x
    for i, L in enumerate(convs):
        cin = widths[i]
        agg = _sc_edge_aggregate(h, src, dst, es[i], cin).reshape(2, N, _WIDE)
        h = _node_mlp(
            h, agg, L['eps'].reshape(1, 1),
            L['W1'].T, L['b1'].reshape(1, -1),
            L['g1'].reshape(1, -1), L['be1'].reshape(1, -1),
            L['W2'].T, L['b2'].reshape(1, -1),
            L['bn_g'].reshape(1, -1), L['bn_b'].reshape(1, -1),
            cin=cin, trailing=(i != len(convs) - 1),
            out_pad=(i != len(convs) - 1),
        )
    M, F = params['mol'], params['final']
    y = _final_mlp(
        h, mol_x,
        M['W1'].T, M['b1'].reshape(1, -1), M['g'].reshape(1, -1),
        M['be'].reshape(1, -1), M['W2'].T, M['b2'].reshape(1, -1),
        F['W1'].T, F['b1'].reshape(1, -1), F['g'].reshape(1, -1),
        F['be'].reshape(1, -1), F['W2'].T, F['b2'].reshape(1, -1),
        ch=convs[-1]['W2'].shape[0],
    )
    return y.reshape(-1)
